# baseline (speedup 1.0000x reference)
import jax
import jax.numpy as jnp
from jax.experimental import pallas as pl
from jax.experimental.pallas import tpu as pltpu

_NC = 1
_PPS = 50


def _kg_kernel(expr_ref, gembT_ref, pemb_ref, mask_ref,
               Wg_ref, bg_ref, WpT_ref, bp_ref,
               A1g_ref, A1pT_ref, a1b_ref, a2_ref, Wo_ref, bo_ref,
               out_ref,
               UT_ref, gsc_ref, cb_ref, L_ref):
    s = pl.program_id(1)
    PPC = cb_ref.shape[0]
    nsteps = PPC // _PPS

    @pl.when(s == 0)
    def _prep():
        H = A1g_ref.shape[0]
        M0 = jnp.concatenate([A1g_ref[...], Wo_ref[...]], axis=0)
        M12 = jnp.dot(M0, Wg_ref[...],
                      preferred_element_type=jnp.float32)
        v12 = jnp.dot(M0, bg_ref[...],
                      preferred_element_type=jnp.float32)
        bw = jax.lax.dot_general(
            M12, gembT_ref[...], (((1,), (1,)), ((), ())),
            preferred_element_type=jnp.float32) + v12
        baseT = bw[:H]
        w0 = bw[H:]
        expr = expr_ref[...]
        UT_ref[...] = (expr[:, None, :] * baseT[None, :, :]).astype(jnp.bfloat16)
        gsc_ref[...] = expr * w0
        p_proj = jnp.dot(pemb_ref[0], WpT_ref[...],
                         preferred_element_type=jnp.float32) + bp_ref[...]
        cb_ref[...] = jnp.dot(p_proj, A1pT_ref[...],
                              preferred_element_type=jnp.float32) + a1b_ref[...]

    @pl.when(s < nsteps)
    def _logits():
        B = UT_ref.shape[0]
        u = UT_ref[...]
        a2r = a2_ref[...].astype(jnp.bfloat16)
        for k in range(_PPS):
            cp = cb_ref[s * _PPS + k, :].astype(jnp.bfloat16)
            t = jnp.tanh(u + cp[None, :, None])
            L_ref[s * _PPS + k] = jnp.concatenate(
                [jnp.dot(a2r, t[b], preferred_element_type=jnp.float32)
                 for b in range(B)], axis=0)

    @pl.when(s == nsteps)
    def _softmax():
        L = L_ref[...]
        valid = (mask_ref[0] > 0.0)[:, None, :]
        Lm = jnp.where(valid, L, jnp.float32(-1e30))
        rowmax = jnp.max(Lm, axis=2, keepdims=True)
        e = jnp.where(valid, jnp.exp(L - rowmax), 0.0)
        denom = jnp.sum(e, axis=2)
        num = jnp.sum(e * gsc_ref[...][None, :, :], axis=2)
        score = jnp.where(denom > 0.0, num / denom + bo_ref[0, 0], 0.0)
        out_ref[...] = score[:, None, :]


def kernel(gene_expression, gene_embeddings, pathway_embeddings,
           gene_pathway_mask, Wg, bg, Wp, bp, A1, a1b, A2, a2b, Wo, bo):
    B, G = gene_expression.shape
    P = pathway_embeddings.shape[0]
    PE = pathway_embeddings.shape[1]
    H = Wg.shape[0]
    G2 = G
    PPC = P // _NC

    expr = gene_expression
    gembT = gene_embeddings
    mask3 = gene_pathway_mask.reshape(_NC, PPC, G2)
    pemb3 = pathway_embeddings.reshape(_NC, PPC, PE)
    A1g = A1[:, :H]
    A1pT = A1[:, H:].T
    WpT = Wp.T
    bg2 = bg.reshape(H, 1)
    bp2 = bp.reshape(1, H)
    a1b2 = a1b.reshape(1, H)
    bo2 = bo.reshape(1, 1)

    def full(x):
        return pl.BlockSpec(x.shape, lambda c, s, _nd=x.ndim: (0,) * _nd)

    def grouped(x):
        return pl.BlockSpec((1,) + x.shape[1:], lambda c, s: (c, 0, 0))

    in_specs = [full(expr), full(gembT), grouped(pemb3), grouped(mask3),
                full(Wg), full(bg2), full(WpT), full(bp2), full(A1g),
                full(A1pT), full(a1b2), full(A2), full(Wo), full(bo2)]

    out = pl.pallas_call(
        _kg_kernel,
        grid=(_NC, PPC // _PPS + 1),
        in_specs=in_specs,
        out_specs=pl.BlockSpec((PPC, 1, B), lambda c, s: (c, 0, 0)),
        out_shape=jax.ShapeDtypeStruct((P, 1, B), jnp.float32),
        scratch_shapes=[
            pltpu.VMEM((B, H, G2), jnp.bfloat16),
            pltpu.VMEM((B, G2), jnp.float32),
            pltpu.VMEM((PPC, H), jnp.float32),
            pltpu.VMEM((PPC, B, G2), jnp.float32),
        ],
        compiler_params=pltpu.CompilerParams(
            dimension_semantics=("parallel", "arbitrary")),
    )(expr, gembT, pemb3, mask3, Wg, bg2, WpT, bp2, A1g, A1pT,
      a1b2, A2, Wo, bo2)
    return out.reshape(P, B).T


# speedup vs baseline: 1.8223x; 1.8223x over previous
import jax
import jax.numpy as jnp
from jax.experimental import pallas as pl
from jax.experimental.pallas import tpu as pltpu

_NC = 1
_PPS = 25


def _kg_kernel(expr_ref, gembT_ref, pemb_ref, mask_ref,
               Wg_ref, bg_ref, WpT_ref, bp_ref,
               A1g_ref, A1pT_ref, a1b_ref, a2_ref, Wo_ref, bo_ref,
               out_ref,
               UT_ref, gsc_ref, cb_ref, L_ref):
    s = pl.program_id(1)
    PPC = cb_ref.shape[0]
    nsteps = PPC // _PPS

    @pl.when(s == 0)
    def _prep():
        H = A1g_ref.shape[0]
        M0 = jnp.concatenate([A1g_ref[...], Wo_ref[...]], axis=0)
        M12 = jnp.dot(M0, Wg_ref[...],
                      preferred_element_type=jnp.float32)
        v12 = jnp.dot(M0, bg_ref[...],
                      preferred_element_type=jnp.float32)
        bw = jax.lax.dot_general(
            M12, gembT_ref[...], (((1,), (1,)), ((), ())),
            preferred_element_type=jnp.float32) + v12
        baseT = bw[:H]
        w0 = bw[H:]
        expr = expr_ref[...]
        UT_ref[...] = (expr[:, None, :] * baseT[None, :, :]).astype(jnp.bfloat16)
        gsc_ref[...] = expr * w0
        p_proj = jnp.dot(pemb_ref[0], WpT_ref[...],
                         preferred_element_type=jnp.float32) + bp_ref[...]
        cb_ref[...] = jnp.dot(p_proj, A1pT_ref[...],
                              preferred_element_type=jnp.float32) + a1b_ref[...]

    @pl.when(s < nsteps)
    def _logits():
        B = UT_ref.shape[0]
        u = UT_ref[...]
        a2r = a2_ref[...].astype(jnp.bfloat16)
        for k in range(_PPS):
            cp = cb_ref[s * _PPS + k, :].astype(jnp.bfloat16)
            t = jnp.tanh(u + cp[None, :, None])
            L_ref[s * _PPS + k] = jnp.concatenate(
                [jnp.dot(a2r, t[b], preferred_element_type=jnp.float32)
                 for b in range(B)], axis=0)

    @pl.when(s == nsteps)
    def _softmax():
        L = L_ref[...]
        valid = (mask_ref[0] > 0.0)[:, None, :]
        Lm = jnp.where(valid, L, jnp.float32(-1e30))
        rowmax = jnp.max(Lm, axis=2, keepdims=True)
        e = jnp.where(valid, jnp.exp(L - rowmax), 0.0)
        denom = jnp.sum(e, axis=2)
        num = jnp.sum(e * gsc_ref[...][None, :, :], axis=2)
        score = jnp.where(denom > 0.0, num / denom + bo_ref[0, 0], 0.0)
        out_ref[...] = score[:, None, :]


def kernel(gene_expression, gene_embeddings, pathway_embeddings,
           gene_pathway_mask, Wg, bg, Wp, bp, A1, a1b, A2, a2b, Wo, bo):
    B, G = gene_expression.shape
    P = pathway_embeddings.shape[0]
    PE = pathway_embeddings.shape[1]
    H = Wg.shape[0]
    G2 = G
    PPC = P // _NC

    expr = gene_expression
    gembT = gene_embeddings
    mask3 = gene_pathway_mask.reshape(_NC, PPC, G2)
    pemb3 = pathway_embeddings.reshape(_NC, PPC, PE)
    A1g = A1[:, :H]
    A1pT = A1[:, H:].T
    WpT = Wp.T
    bg2 = bg.reshape(H, 1)
    bp2 = bp.reshape(1, H)
    a1b2 = a1b.reshape(1, H)
    bo2 = bo.reshape(1, 1)

    def full(x):
        return pl.BlockSpec(x.shape, lambda c, s, _nd=x.ndim: (0,) * _nd)

    def grouped(x):
        return pl.BlockSpec((1,) + x.shape[1:], lambda c, s: (c, 0, 0))

    in_specs = [full(expr), full(gembT), grouped(pemb3), grouped(mask3),
                full(Wg), full(bg2), full(WpT), full(bp2), full(A1g),
                full(A1pT), full(a1b2), full(A2), full(Wo), full(bo2)]

    out = pl.pallas_call(
        _kg_kernel,
        grid=(_NC, PPC // _PPS + 1),
        in_specs=in_specs,
        out_specs=pl.BlockSpec((PPC, 1, B), lambda c, s: (c, 0, 0)),
        out_shape=jax.ShapeDtypeStruct((P, 1, B), jnp.float32),
        scratch_shapes=[
            pltpu.VMEM((B, H, G2), jnp.bfloat16),
            pltpu.VMEM((B, G2), jnp.float32),
            pltpu.VMEM((PPC, H), jnp.float32),
            pltpu.VMEM((PPC, B, G2), jnp.float32),
        ],
        compiler_params=pltpu.CompilerParams(
            dimension_semantics=("parallel", "arbitrary")),
    )(expr, gembT, pemb3, mask3, Wg, bg2, WpT, bp2, A1g, A1pT,
      a1b2, A2, Wo, bo2)
    return out.reshape(P, B).T
